# baseline (speedup 1.0000x reference)
import functools

import jax
import jax.numpy as jnp
from jax import lax
from jax.experimental import pallas as pl
from jax.experimental.pallas import tpu as pltpu
from jax.experimental.pallas import tpu_sc as plsc

N = 10000
E = 320000
D_IN = 128
H1 = 256
H2 = 32
C = 16

NC, NS = 2, 16
NW = NC * NS
CH = 128
K = 80
EP = NW * K * CH
NP = 10240
RPT = NP // NS
RB = 5000


def _make_pass(w, ge, dt, nb, split, hbm_every=0):
  ept = EP // (NS if split else NW)
  rept = E // (NS if split else NW)
  padt = ept - rept
  ng = ept // ge
  nsc = ge // CH
  assert ept % ge == 0 and ng >= nb and rept % 8 == 0 and padt % 8 == 0
  mesh = plsc.VectorSubcoreMesh(
      core_axis_name="c", subcore_axis_name="s",
      num_cores=NC, num_subcores=NS)

  @functools.partial(
      pl.kernel,
      out_type=jax.ShapeDtypeStruct((NC, NP, w), dt),
      mesh=mesh,
      compiler_params=pltpu.CompilerParams(use_tc_tiling_on_sc=False),
      scratch_types=[
          pltpu.VMEM((ept,), jnp.int32),
          pltpu.VMEM((ept,), jnp.int32),
          pltpu.VMEM((nb, ge, w), dt),
          pltpu.VMEM_SHARED((NP, w), dt),
          pltpu.VMEM_SHARED((NP, w), dt),
          [pltpu.SemaphoreType.DMA] * nb,
          [pltpu.SemaphoreType.DMA] * nb,
      ],
  )
  def agg(g_hbm, src_hbm, dst_hbm, psrc_hbm, pdst_hbm, z_hbm, out_hbm,
          src_v, dst_v, rows_v, acc, g_s, gsems, ssems):
    c = lax.axis_index("c")
    s = lax.axis_index("s")
    nrt = N // NS
    wid = s if split else s * NC + c
    pltpu.sync_copy(z_hbm.at[pl.ds(s * RPT, RPT)], acc.at[pl.ds(s * RPT, RPT)])
    if split:
      pltpu.sync_copy(g_hbm.at[c].at[pl.ds(s * nrt, nrt)],
                      g_s.at[pl.ds(s * nrt, nrt)])
    else:
      pltpu.sync_copy(g_hbm.at[pl.ds(s * nrt, nrt)],
                      g_s.at[pl.ds(s * nrt, nrt)])
    pltpu.sync_copy(src_hbm.at[pl.ds(wid * rept, rept)],
                    src_v.at[pl.ds(0, rept)])
    pltpu.sync_copy(dst_hbm.at[pl.ds(wid * rept, rept)],
                    dst_v.at[pl.ds(0, rept)])
    pltpu.sync_copy(psrc_hbm.at[pl.ds(0, padt)], src_v.at[pl.ds(rept, padt)])
    pltpu.sync_copy(pdst_hbm.at[pl.ds(0, padt)], dst_v.at[pl.ds(rept, padt)])
    plsc.subcore_barrier()

    def gdesc(j, b, make):
      f = pltpu.make_async_copy if make else pltpu.async_copy
      if hbm_every and j % hbm_every == 0:
        src_tbl = g_hbm.at[c] if split else g_hbm
      else:
        src_tbl = g_s
      return f(src_tbl.at[src_v.at[pl.ds(j * ge, ge)]], rows_v.at[b],
               gsems[b])

    def sdesc(j, q, b, make):
      dsl = dst_v.at[pl.ds((j * nsc + q) * CH, CH)]
      if make:
        return pltpu.make_async_copy(rows_v.at[b].at[pl.ds(q * CH, CH)],
                                     acc.at[dsl], ssems[b])
      return pltpu.async_copy(rows_v.at[b].at[pl.ds(q * CH, CH)],
                              acc.at[dsl], ssems[b], add=True)

    for j in range(nb):
      gdesc(j, j % nb, False)
    for j in range(ng):
      b = j % nb
      gdesc(j, b, True).wait()
      for q in range(nsc):
        sdesc(j, q, b, False)
      for q in range(nsc):
        sdesc(j, q, b, True).wait()
      if j + nb < ng:
        gdesc(j + nb, b, False)

    plsc.subcore_barrier()
    pltpu.sync_copy(acc.at[pl.ds(s * RPT, RPT)],
                    out_hbm.at[c].at[pl.ds(s * RPT, RPT)])

  return agg


def _make_deg():
  mesh = plsc.VectorSubcoreMesh(
      core_axis_name="c", subcore_axis_name="s",
      num_cores=NC, num_subcores=NS)

  @functools.partial(
      pl.kernel,
      out_type=jax.ShapeDtypeStruct((NC, NP, C), jnp.float32),
      mesh=mesh,
      compiler_params=pltpu.CompilerParams(use_tc_tiling_on_sc=False),
      scratch_types=[
          pltpu.VMEM((EP // NW,), jnp.int32),
          pltpu.VMEM((CH, C), jnp.float32),
          pltpu.VMEM_SHARED((NP, C), jnp.float32),
      ],
  )
  def deg(ones_hbm, dst_hbm, pdst_hbm, z_hbm, out_hbm, dst_v, ones_v, acc):
    c = lax.axis_index("c")
    s = lax.axis_index("s")
    wid = s * NC + c
    rept = E // NW
    padt = EP // NW - rept
    pltpu.sync_copy(z_hbm.at[pl.ds(s * RPT, RPT)], acc.at[pl.ds(s * RPT, RPT)])
    pltpu.sync_copy(ones_hbm, ones_v)
    pltpu.sync_copy(dst_hbm.at[pl.ds(wid * rept, rept)],
                    dst_v.at[pl.ds(0, rept)])
    pltpu.sync_copy(pdst_hbm.at[pl.ds(0, padt)], dst_v.at[pl.ds(rept, padt)])
    plsc.subcore_barrier()

    def body(j, carry):
      pltpu.sync_copy(ones_v, acc.at[dst_v.at[pl.ds(j * CH, CH)]], add=True)
      return carry

    lax.fori_loop(0, K, body, 0)
    plsc.subcore_barrier()
    pltpu.sync_copy(acc.at[pl.ds(s * RPT, RPT)],
                    out_hbm.at[c].at[pl.ds(s * RPT, RPT)])

  return deg


_agg128 = _make_pass(D_IN // 2, 512, jnp.bfloat16, 2, True, hbm_every=3)
_agg32 = _make_pass(H2, 2048, jnp.bfloat16, 2, False)
_agg16 = _make_pass(C, 2048, jnp.float32, 2, False)
_deg16 = _make_deg()


def _tcA_body(degp_ref, x_ref, dinv_ref, g0cat_ref):
  deg = degp_ref[0][:, 0:1] + degp_ref[1][:, 0:1] + 1.0
  dinv = lax.rsqrt(deg)
  dinv_ref[...] = dinv
  g0 = x_ref[...] * dinv
  g0cat_ref[0] = g0[:, :D_IN // 2].astype(jnp.bfloat16)
  g0cat_ref[1] = g0[:, D_IN // 2:].astype(jnp.bfloat16)


def _tcB_body(p_ref, g0cat_ref, dinv_ref, w1_ref, b1_ref, w2_ref, g2_ref):
  dinv = dinv_ref[...]
  pcat = jnp.concatenate([p_ref[0], p_ref[1]], axis=1).astype(jnp.float32)
  gcat = jnp.concatenate([g0cat_ref[0], g0cat_ref[1]],
                         axis=1).astype(jnp.float32)
  a1 = dinv * (pcat + gcat)
  h1 = jnp.dot(a1, w1_ref[...], preferred_element_type=jnp.float32)
  h1 = jnp.maximum(h1 + b1_ref[...], 0.0)
  g2 = dinv * jnp.dot(h1, w2_ref[...], preferred_element_type=jnp.float32)
  g2_ref[...] = g2.astype(jnp.bfloat16)


def _tcC_body(p_ref, g2_ref, dinv_ref, b2_ref, w3_ref, g3_ref):
  dinv = dinv_ref[...]
  p01 = (p_ref[0] + p_ref[1]).astype(jnp.float32)
  a2 = dinv * (p01 + g2_ref[...].astype(jnp.float32))
  h2 = jnp.maximum(a2 + b2_ref[...], 0.0)
  g3_ref[...] = dinv * jnp.dot(h2, w3_ref[...],
                               preferred_element_type=jnp.float32)


def _tcD_body(p_ref, g3_ref, dinv_ref, b3_ref, out_ref):
  dinv = dinv_ref[...]
  out_ref[...] = dinv * (p_ref[0] + p_ref[1] + g3_ref[...]) + b3_ref[...]


def _row_spec(w):
  return pl.BlockSpec((RB, w), lambda b: (b, 0))


def _p_spec(w):
  return pl.BlockSpec((NC, RB, w), lambda b: (0, b, 0))


def _full_spec(r, w):
  return pl.BlockSpec((r, w), lambda b: (0, 0))


def _tcA(degp, x):
  return pl.pallas_call(
      _tcA_body,
      grid=(N // RB,),
      in_specs=[_p_spec(C), _row_spec(D_IN)],
      out_specs=[_row_spec(1), _p_spec(D_IN // 2)],
      out_shape=[jax.ShapeDtypeStruct((N, 1), jnp.float32),
                 jax.ShapeDtypeStruct((NC, N, D_IN // 2), jnp.bfloat16)],
  )(degp, x)


def _tcB(p1, g0cat, dinv, W1, b1, W2):
  return pl.pallas_call(
      _tcB_body,
      grid=(N // RB,),
      in_specs=[_p_spec(D_IN // 2), _p_spec(D_IN // 2), _row_spec(1),
                _full_spec(D_IN, H1), _full_spec(1, H1), _full_spec(H1, H2)],
      out_specs=_row_spec(H2),
      out_shape=jax.ShapeDtypeStruct((N, H2), jnp.bfloat16),
  )(p1, g0cat, dinv, W1, b1.reshape(1, H1), W2)


def _tcC(p2, g2, dinv, b2, W3):
  return pl.pallas_call(
      _tcC_body,
      grid=(N // RB,),
      in_specs=[_p_spec(H2), _row_spec(H2), _row_spec(1),
                _full_spec(1, H2), _full_spec(H2, C)],
      out_specs=_row_spec(C),
      out_shape=jax.ShapeDtypeStruct((N, C), jnp.float32),
  )(p2, g2, dinv, b2.reshape(1, H2), W3)


def _tcD(p3, g3, dinv, b3):
  return pl.pallas_call(
      _tcD_body,
      grid=(N // RB,),
      in_specs=[_p_spec(C), _row_spec(C), _row_spec(1), _full_spec(1, C)],
      out_specs=_row_spec(C),
      out_shape=jax.ShapeDtypeStruct((N, C), jnp.float32),
  )(p3, g3, dinv, b3.reshape(1, C))


def kernel(x, edge_index, W1, b1, W2, b2, W3, b3):
  src = edge_index[0].astype(jnp.int32)
  dst = edge_index[1].astype(jnp.int32)
  mpad = EP // NS - E // NS
  psrc = jnp.zeros((mpad,), jnp.int32)
  pdst = N + jnp.arange(mpad, dtype=jnp.int32) % (NP - N - 8)

  z16 = jnp.zeros((NP, C), jnp.float32)
  z32 = jnp.zeros((NP, H2), jnp.bfloat16)
  z64 = jnp.zeros((NP, D_IN // 2), jnp.bfloat16)
  ones = jnp.ones((CH, C), jnp.float32)

  degp = _deg16(ones, dst, pdst, z16)
  dinv, g0cat = _tcA(degp, x)
  p1 = _agg128(g0cat, src, dst, psrc, pdst, z64)
  g2 = _tcB(p1, g0cat, dinv, W1, b1, W2)
  p2 = _agg32(g2, src, dst, psrc, pdst, z32)
  g3 = _tcC(p2, g2, dinv, b2, W3)
  p3 = _agg16(g3, src, dst, psrc, pdst, z16)
  return _tcD(p3, g3, dinv, b3)


# speedup vs baseline: 1.2401x; 1.2401x over previous
import functools

import jax
import jax.numpy as jnp
from jax import lax
from jax.experimental import pallas as pl
from jax.experimental.pallas import tpu as pltpu
from jax.experimental.pallas import tpu_sc as plsc

N = 10000
E = 320000
D_IN = 128
H1 = 256
H2 = 32
C = 16

NC, NS = 2, 16
NW = NC * NS
CH = 128
K = 80
EP = NW * K * CH
NP = 10240
RPT = NP // NS
RB = 5000


def _make_pass(w, ge, dt, nb, split, hbm_every=0):
  ept = EP // (NS if split else NW)
  rept = E // (NS if split else NW)
  padt = ept - rept
  ng = ept // ge
  nsc = ge // CH
  assert ept % ge == 0 and ng >= nb and rept % 8 == 0 and padt % 8 == 0
  mesh = plsc.VectorSubcoreMesh(
      core_axis_name="c", subcore_axis_name="s",
      num_cores=NC, num_subcores=NS)

  @functools.partial(
      pl.kernel,
      out_type=jax.ShapeDtypeStruct((NC, NP, w), dt),
      mesh=mesh,
      compiler_params=pltpu.CompilerParams(use_tc_tiling_on_sc=False),
      scratch_types=[
          pltpu.VMEM((ept,), jnp.int32),
          pltpu.VMEM((ept,), jnp.int32),
          pltpu.VMEM((nb, ge, w), dt),
          pltpu.VMEM_SHARED((NP, w), dt),
          pltpu.VMEM_SHARED((N, w), dt),
          [pltpu.SemaphoreType.DMA] * nb,
          [pltpu.SemaphoreType.DMA] * nb,
      ],
  )
  def agg(g_hbm, src_hbm, dst_hbm, psrc_hbm, pdst_hbm, z_hbm, out_hbm,
          src_v, dst_v, rows_v, acc, g_s, gsems, ssems):
    c = lax.axis_index("c")
    s = lax.axis_index("s")
    nrt = N // NS
    wid = s if split else s * NC + c
    pltpu.sync_copy(z_hbm.at[pl.ds(s * RPT, RPT)], acc.at[pl.ds(s * RPT, RPT)])
    if split:
      pltpu.sync_copy(g_hbm.at[c].at[pl.ds(s * nrt, nrt)],
                      g_s.at[pl.ds(s * nrt, nrt)])
    else:
      pltpu.sync_copy(g_hbm.at[pl.ds(s * nrt, nrt)],
                      g_s.at[pl.ds(s * nrt, nrt)])
    pltpu.sync_copy(src_hbm.at[pl.ds(wid * rept, rept)],
                    src_v.at[pl.ds(0, rept)])
    pltpu.sync_copy(dst_hbm.at[pl.ds(wid * rept, rept)],
                    dst_v.at[pl.ds(0, rept)])
    pltpu.sync_copy(psrc_hbm.at[pl.ds(0, padt)], src_v.at[pl.ds(rept, padt)])
    pltpu.sync_copy(pdst_hbm.at[pl.ds(0, padt)], dst_v.at[pl.ds(rept, padt)])
    plsc.subcore_barrier()

    def gdesc(j, b, make):
      f = pltpu.make_async_copy if make else pltpu.async_copy
      if hbm_every and j % hbm_every == 0:
        src_tbl = g_hbm.at[c] if split else g_hbm
      else:
        src_tbl = g_s
      return f(src_tbl.at[src_v.at[pl.ds(j * ge, ge)]], rows_v.at[b],
               gsems[b])

    def sdesc(j, q, b, make):
      dsl = dst_v.at[pl.ds((j * nsc + q) * CH, CH)]
      if make:
        return pltpu.make_async_copy(rows_v.at[b].at[pl.ds(q * CH, CH)],
                                     acc.at[dsl], ssems[b])
      return pltpu.async_copy(rows_v.at[b].at[pl.ds(q * CH, CH)],
                              acc.at[dsl], ssems[b], add=True)

    for j in range(nb):
      gdesc(j, j % nb, False)
    for j in range(ng):
      b = j % nb
      gdesc(j, b, True).wait()
      for q in range(nsc):
        sdesc(j, q, b, False)
      for q in range(nsc):
        sdesc(j, q, b, True).wait()
      if j + nb < ng:
        gdesc(j + nb, b, False)

    plsc.subcore_barrier()
    pltpu.sync_copy(acc.at[pl.ds(s * RPT, RPT)],
                    out_hbm.at[c].at[pl.ds(s * RPT, RPT)])

  return agg


def _make_deg():
  mesh = plsc.VectorSubcoreMesh(
      core_axis_name="c", subcore_axis_name="s",
      num_cores=NC, num_subcores=NS)

  @functools.partial(
      pl.kernel,
      out_type=jax.ShapeDtypeStruct((NC, NP, C), jnp.float32),
      mesh=mesh,
      compiler_params=pltpu.CompilerParams(use_tc_tiling_on_sc=False),
      scratch_types=[
          pltpu.VMEM((EP // NW,), jnp.int32),
          pltpu.VMEM((CH, C), jnp.float32),
          pltpu.VMEM_SHARED((NP, C), jnp.float32),
      ],
  )
  def deg(ones_hbm, dst_hbm, pdst_hbm, z_hbm, out_hbm, dst_v, ones_v, acc):
    c = lax.axis_index("c")
    s = lax.axis_index("s")
    wid = s * NC + c
    rept = E // NW
    padt = EP // NW - rept
    pltpu.sync_copy(z_hbm.at[pl.ds(s * RPT, RPT)], acc.at[pl.ds(s * RPT, RPT)])
    pltpu.sync_copy(ones_hbm, ones_v)
    pltpu.sync_copy(dst_hbm.at[pl.ds(wid * rept, rept)],
                    dst_v.at[pl.ds(0, rept)])
    pltpu.sync_copy(pdst_hbm.at[pl.ds(0, padt)], dst_v.at[pl.ds(rept, padt)])
    plsc.subcore_barrier()

    def body(j, carry):
      pltpu.sync_copy(ones_v, acc.at[dst_v.at[pl.ds(j * CH, CH)]], add=True)
      return carry

    lax.fori_loop(0, K, body, 0)
    plsc.subcore_barrier()
    pltpu.sync_copy(acc.at[pl.ds(s * RPT, RPT)],
                    out_hbm.at[c].at[pl.ds(s * RPT, RPT)])

  return deg


_agg128 = _make_pass(D_IN // 2, 512, jnp.bfloat16, 3, True, hbm_every=2)
_agg32 = _make_pass(H2, 2048, jnp.bfloat16, 2, False)
_agg16 = _make_pass(C, 2048, jnp.float32, 2, False)
_deg16 = _make_deg()


def _tcA_body(degp_ref, x_ref, dinv_ref, g0cat_ref):
  deg = degp_ref[0][:, 0:1] + degp_ref[1][:, 0:1] + 1.0
  dinv = lax.rsqrt(deg)
  dinv_ref[...] = dinv
  g0 = x_ref[...] * dinv
  g0cat_ref[0] = g0[:, :D_IN // 2].astype(jnp.bfloat16)
  g0cat_ref[1] = g0[:, D_IN // 2:].astype(jnp.bfloat16)


def _tcB_body(p_ref, g0cat_ref, dinv_ref, w1_ref, b1_ref, w2_ref, g2_ref):
  dinv = dinv_ref[...]
  pcat = jnp.concatenate([p_ref[0], p_ref[1]], axis=1).astype(jnp.float32)
  gcat = jnp.concatenate([g0cat_ref[0], g0cat_ref[1]],
                         axis=1).astype(jnp.float32)
  a1 = dinv * (pcat + gcat)
  h1 = jnp.dot(a1.astype(jnp.bfloat16), w1_ref[...].astype(jnp.bfloat16),
               preferred_element_type=jnp.float32)
  h1 = jnp.maximum(h1 + b1_ref[...], 0.0)
  g2 = dinv * jnp.dot(h1.astype(jnp.bfloat16),
                      w2_ref[...].astype(jnp.bfloat16),
                      preferred_element_type=jnp.float32)
  g2_ref[...] = g2.astype(jnp.bfloat16)


def _tcC_body(p_ref, g2_ref, dinv_ref, b2_ref, w3_ref, g3_ref):
  dinv = dinv_ref[...]
  p01 = (p_ref[0] + p_ref[1]).astype(jnp.float32)
  a2 = dinv * (p01 + g2_ref[...].astype(jnp.float32))
  h2 = jnp.maximum(a2 + b2_ref[...], 0.0)
  g3_ref[...] = dinv * jnp.dot(h2, w3_ref[...],
                               preferred_element_type=jnp.float32)


def _tcD_body(p_ref, g3_ref, dinv_ref, b3_ref, out_ref):
  dinv = dinv_ref[...]
  out_ref[...] = dinv * (p_ref[0] + p_ref[1] + g3_ref[...]) + b3_ref[...]


def _row_spec(w):
  return pl.BlockSpec((RB, w), lambda b: (b, 0))


def _p_spec(w):
  return pl.BlockSpec((NC, RB, w), lambda b: (0, b, 0))


def _full_spec(r, w):
  return pl.BlockSpec((r, w), lambda b: (0, 0))


def _tcA(degp, x):
  return pl.pallas_call(
      _tcA_body,
      grid=(N // RB,),
      in_specs=[_p_spec(C), _row_spec(D_IN)],
      out_specs=[_row_spec(1), _p_spec(D_IN // 2)],
      out_shape=[jax.ShapeDtypeStruct((N, 1), jnp.float32),
                 jax.ShapeDtypeStruct((NC, N, D_IN // 2), jnp.bfloat16)],
  )(degp, x)


def _tcB(p1, g0cat, dinv, W1, b1, W2):
  return pl.pallas_call(
      _tcB_body,
      grid=(N // RB,),
      in_specs=[_p_spec(D_IN // 2), _p_spec(D_IN // 2), _row_spec(1),
                _full_spec(D_IN, H1), _full_spec(1, H1), _full_spec(H1, H2)],
      out_specs=_row_spec(H2),
      out_shape=jax.ShapeDtypeStruct((N, H2), jnp.bfloat16),
  )(p1, g0cat, dinv, W1, b1.reshape(1, H1), W2)


def _tcC(p2, g2, dinv, b2, W3):
  return pl.pallas_call(
      _tcC_body,
      grid=(N // RB,),
      in_specs=[_p_spec(H2), _row_spec(H2), _row_spec(1),
                _full_spec(1, H2), _full_spec(H2, C)],
      out_specs=_row_spec(C),
      out_shape=jax.ShapeDtypeStruct((N, C), jnp.float32),
  )(p2, g2, dinv, b2.reshape(1, H2), W3)


def _tcD(p3, g3, dinv, b3):
  return pl.pallas_call(
      _tcD_body,
      grid=(N // RB,),
      in_specs=[_p_spec(C), _row_spec(C), _row_spec(1), _full_spec(1, C)],
      out_specs=_row_spec(C),
      out_shape=jax.ShapeDtypeStruct((N, C), jnp.float32),
  )(p3, g3, dinv, b3.reshape(1, C))


def kernel(x, edge_index, W1, b1, W2, b2, W3, b3):
  src = edge_index[0].astype(jnp.int32)
  dst = edge_index[1].astype(jnp.int32)
  mpad = EP // NS - E // NS
  psrc = jnp.zeros((mpad,), jnp.int32)
  pdst = N + jnp.arange(mpad, dtype=jnp.int32) % (NP - N - 8)

  z16 = jnp.zeros((NP, C), jnp.float32)
  z32 = jnp.zeros((NP, H2), jnp.bfloat16)
  z64 = jnp.zeros((NP, D_IN // 2), jnp.bfloat16)
  ones = jnp.ones((CH, C), jnp.float32)

  degp = _deg16(ones, dst, pdst, z16)
  dinv, g0cat = _tcA(degp, x)
  p1 = _agg128(g0cat, src, dst, psrc, pdst, z64)
  g2 = _tcB(p1, g0cat, dinv, W1, b1, W2)
  p2 = _agg32(g2, src, dst, psrc, pdst, z32)
  g3 = _tcC(p2, g2, dinv, b2, W3)
  p3 = _agg16(g3, src, dst, psrc, pdst, z16)
  return _tcD(p3, g3, dinv, b3)


# speedup vs baseline: 1.2970x; 1.0459x over previous
import functools

import jax
import jax.numpy as jnp
from jax import lax
from jax.experimental import pallas as pl
from jax.experimental.pallas import tpu as pltpu
from jax.experimental.pallas import tpu_sc as plsc

N = 10000
E = 320000
D_IN = 128
H1 = 256
H2 = 32
C = 16

NC, NS = 2, 16
NW = NC * NS
CH = 128
K = 80
EP = NW * K * CH
NP = 10240
RPT = NP // NS
RB = 5000


def _make_pass(w, ge, dt, nb, split, hbm_every=0):
  ept = EP // (NS if split else NW)
  rept = E // (NS if split else NW)
  padt = ept - rept
  ng = ept // ge
  nsc = ge // CH
  assert ept % ge == 0 and ng >= nb and rept % 8 == 0 and padt % 8 == 0
  mesh = plsc.VectorSubcoreMesh(
      core_axis_name="c", subcore_axis_name="s",
      num_cores=NC, num_subcores=NS)

  @functools.partial(
      pl.kernel,
      out_type=jax.ShapeDtypeStruct((NC, NP, w), dt),
      mesh=mesh,
      compiler_params=pltpu.CompilerParams(use_tc_tiling_on_sc=False),
      scratch_types=[
          pltpu.VMEM((ept,), jnp.int32),
          pltpu.VMEM((ept,), jnp.int32),
          pltpu.VMEM((nb, ge, w), dt),
          pltpu.VMEM_SHARED((NP, w), dt),
          pltpu.VMEM_SHARED((N, w), dt),
          [pltpu.SemaphoreType.DMA] * nb,
          [pltpu.SemaphoreType.DMA] * nb,
      ],
  )
  def agg(g_hbm, src_hbm, dst_hbm, psrc_hbm, pdst_hbm, z_hbm, out_hbm,
          src_v, dst_v, rows_v, acc, g_s, gsems, ssems):
    c = lax.axis_index("c")
    s = lax.axis_index("s")
    nrt = N // NS
    wid = s if split else s * NC + c
    pltpu.sync_copy(z_hbm.at[pl.ds(s * RPT, RPT)], acc.at[pl.ds(s * RPT, RPT)])
    if split:
      pltpu.sync_copy(g_hbm.at[c].at[pl.ds(s * nrt, nrt)],
                      g_s.at[pl.ds(s * nrt, nrt)])
    else:
      pltpu.sync_copy(g_hbm.at[pl.ds(s * nrt, nrt)],
                      g_s.at[pl.ds(s * nrt, nrt)])
    pltpu.sync_copy(src_hbm.at[pl.ds(wid * rept, rept)],
                    src_v.at[pl.ds(0, rept)])
    pltpu.sync_copy(dst_hbm.at[pl.ds(wid * rept, rept)],
                    dst_v.at[pl.ds(0, rept)])
    pltpu.sync_copy(psrc_hbm.at[pl.ds(0, padt)], src_v.at[pl.ds(rept, padt)])
    pltpu.sync_copy(pdst_hbm.at[pl.ds(0, padt)], dst_v.at[pl.ds(rept, padt)])
    plsc.subcore_barrier()

    def gdesc(j, b, make):
      f = pltpu.make_async_copy if make else pltpu.async_copy
      if hbm_every and j % hbm_every == 0:
        src_tbl = g_hbm.at[c] if split else g_hbm
      else:
        src_tbl = g_s
      return f(src_tbl.at[src_v.at[pl.ds(j * ge, ge)]], rows_v.at[b],
               gsems[b])

    def sdesc(j, q, b, make):
      dsl = dst_v.at[pl.ds((j * nsc + q) * CH, CH)]
      if make:
        return pltpu.make_async_copy(rows_v.at[b].at[pl.ds(q * CH, CH)],
                                     acc.at[dsl], ssems[b])
      return pltpu.async_copy(rows_v.at[b].at[pl.ds(q * CH, CH)],
                              acc.at[dsl], ssems[b], add=True)

    for j in range(nb):
      gdesc(j, j % nb, False)
    for j in range(ng):
      b = j % nb
      gdesc(j, b, True).wait()
      for q in range(nsc):
        sdesc(j, q, b, False)
      for q in range(nsc):
        sdesc(j, q, b, True).wait()
      if j + nb < ng:
        gdesc(j + nb, b, False)

    plsc.subcore_barrier()
    pltpu.sync_copy(acc.at[pl.ds(s * RPT, RPT)],
                    out_hbm.at[c].at[pl.ds(s * RPT, RPT)])

  return agg


def _make_deg():
  mesh = plsc.VectorSubcoreMesh(
      core_axis_name="c", subcore_axis_name="s",
      num_cores=NC, num_subcores=NS)

  @functools.partial(
      pl.kernel,
      out_type=jax.ShapeDtypeStruct((NC, NP, C), jnp.float32),
      mesh=mesh,
      compiler_params=pltpu.CompilerParams(use_tc_tiling_on_sc=False),
      scratch_types=[
          pltpu.VMEM((EP // NW,), jnp.int32),
          pltpu.VMEM((CH, C), jnp.float32),
          pltpu.VMEM_SHARED((NP, C), jnp.float32),
      ],
  )
  def deg(ones_hbm, dst_hbm, pdst_hbm, z_hbm, out_hbm, dst_v, ones_v, acc):
    c = lax.axis_index("c")
    s = lax.axis_index("s")
    wid = s * NC + c
    rept = E // NW
    padt = EP // NW - rept
    pltpu.sync_copy(z_hbm.at[pl.ds(s * RPT, RPT)], acc.at[pl.ds(s * RPT, RPT)])
    pltpu.sync_copy(ones_hbm, ones_v)
    pltpu.sync_copy(dst_hbm.at[pl.ds(wid * rept, rept)],
                    dst_v.at[pl.ds(0, rept)])
    pltpu.sync_copy(pdst_hbm.at[pl.ds(0, padt)], dst_v.at[pl.ds(rept, padt)])
    plsc.subcore_barrier()

    def body(j, carry):
      pltpu.sync_copy(ones_v, acc.at[dst_v.at[pl.ds(j * CH, CH)]], add=True)
      return carry

    lax.fori_loop(0, K, body, 0)
    plsc.subcore_barrier()
    pltpu.sync_copy(acc.at[pl.ds(s * RPT, RPT)],
                    out_hbm.at[c].at[pl.ds(s * RPT, RPT)])

  return deg


_agg128 = _make_pass(D_IN // 2, 512, jnp.bfloat16, 2, True, hbm_every=2)
_agg32 = _make_pass(H2, 2048, jnp.bfloat16, 2, False)
_agg16 = _make_pass(C, 2048, jnp.float32, 2, False)
_deg16 = _make_deg()


def _tcA_body(degp_ref, x_ref, dinv_ref, g0cat_ref):
  deg = degp_ref[0][:, 0:1] + degp_ref[1][:, 0:1] + 1.0
  dinv = lax.rsqrt(deg)
  dinv_ref[...] = dinv
  g0 = x_ref[...] * dinv
  g0cat_ref[0] = g0[:, :D_IN // 2].astype(jnp.bfloat16)
  g0cat_ref[1] = g0[:, D_IN // 2:].astype(jnp.bfloat16)


def _tcB_body(p_ref, g0cat_ref, dinv_ref, w1_ref, b1_ref, w2_ref, g2_ref):
  dinv = dinv_ref[...]
  pcat = jnp.concatenate([p_ref[0], p_ref[1]], axis=1).astype(jnp.float32)
  gcat = jnp.concatenate([g0cat_ref[0], g0cat_ref[1]],
                         axis=1).astype(jnp.float32)
  a1 = dinv * (pcat + gcat)
  h1 = jnp.dot(a1.astype(jnp.bfloat16), w1_ref[...].astype(jnp.bfloat16),
               preferred_element_type=jnp.float32)
  h1 = jnp.maximum(h1 + b1_ref[...], 0.0)
  g2 = dinv * jnp.dot(h1.astype(jnp.bfloat16),
                      w2_ref[...].astype(jnp.bfloat16),
                      preferred_element_type=jnp.float32)
  g2_ref[...] = g2.astype(jnp.bfloat16)


def _tcC_body(p_ref, g2_ref, dinv_ref, b2_ref, w3_ref, g3_ref):
  dinv = dinv_ref[...]
  p01 = (p_ref[0] + p_ref[1]).astype(jnp.float32)
  a2 = dinv * (p01 + g2_ref[...].astype(jnp.float32))
  h2 = jnp.maximum(a2 + b2_ref[...], 0.0)
  g3_ref[...] = dinv * jnp.dot(h2, w3_ref[...],
                               preferred_element_type=jnp.float32)


def _tcD_body(p_ref, g3_ref, dinv_ref, b3_ref, out_ref):
  dinv = dinv_ref[...]
  out_ref[...] = dinv * (p_ref[0] + p_ref[1] + g3_ref[...]) + b3_ref[...]


def _row_spec(w):
  return pl.BlockSpec((RB, w), lambda b: (b, 0))


def _p_spec(w):
  return pl.BlockSpec((NC, RB, w), lambda b: (0, b, 0))


def _full_spec(r, w):
  return pl.BlockSpec((r, w), lambda b: (0, 0))


def _tcA(degp, x):
  return pl.pallas_call(
      _tcA_body,
      grid=(N // RB,),
      in_specs=[_p_spec(C), _row_spec(D_IN)],
      out_specs=[_row_spec(1), _p_spec(D_IN // 2)],
      out_shape=[jax.ShapeDtypeStruct((N, 1), jnp.float32),
                 jax.ShapeDtypeStruct((NC, N, D_IN // 2), jnp.bfloat16)],
  )(degp, x)


def _tcB(p1, g0cat, dinv, W1, b1, W2):
  return pl.pallas_call(
      _tcB_body,
      grid=(N // RB,),
      in_specs=[_p_spec(D_IN // 2), _p_spec(D_IN // 2), _row_spec(1),
                _full_spec(D_IN, H1), _full_spec(1, H1), _full_spec(H1, H2)],
      out_specs=_row_spec(H2),
      out_shape=jax.ShapeDtypeStruct((N, H2), jnp.bfloat16),
  )(p1, g0cat, dinv, W1, b1.reshape(1, H1), W2)


def _tcC(p2, g2, dinv, b2, W3):
  return pl.pallas_call(
      _tcC_body,
      grid=(N // RB,),
      in_specs=[_p_spec(H2), _row_spec(H2), _row_spec(1),
                _full_spec(1, H2), _full_spec(H2, C)],
      out_specs=_row_spec(C),
      out_shape=jax.ShapeDtypeStruct((N, C), jnp.float32),
  )(p2, g2, dinv, b2.reshape(1, H2), W3)


def _tcD(p3, g3, dinv, b3):
  return pl.pallas_call(
      _tcD_body,
      grid=(N // RB,),
      in_specs=[_p_spec(C), _row_spec(C), _row_spec(1), _full_spec(1, C)],
      out_specs=_row_spec(C),
      out_shape=jax.ShapeDtypeStruct((N, C), jnp.float32),
  )(p3, g3, dinv, b3.reshape(1, C))


def kernel(x, edge_index, W1, b1, W2, b2, W3, b3):
  src = edge_index[0].astype(jnp.int32)
  dst = edge_index[1].astype(jnp.int32)
  mpad = EP // NS - E // NS
  psrc = jnp.zeros((mpad,), jnp.int32)
  pdst = N + jnp.arange(mpad, dtype=jnp.int32) % (NP - N - 8)

  z16 = jnp.zeros((NP, C), jnp.float32)
  z32 = jnp.zeros((NP, H2), jnp.bfloat16)
  z64 = jnp.zeros((NP, D_IN // 2), jnp.bfloat16)
  ones = jnp.ones((CH, C), jnp.float32)

  degp = _deg16(ones, dst, pdst, z16)
  dinv, g0cat = _tcA(degp, x)
  p1 = _agg128(g0cat, src, dst, psrc, pdst, z64)
  g2 = _tcB(p1, g0cat, dinv, W1, b1, W2)
  p2 = _agg32(g2, src, dst, psrc, pdst, z32)
  g3 = _tcC(p2, g2, dinv, b2, W3)
  p3 = _agg16(g3, src, dst, psrc, pdst, z16)
  return _tcD(p3, g3, dinv, b3)
